# Initial kernel scaffold; baseline (speedup 1.0000x reference)
import jax
import jax.numpy as jnp
from jax.experimental import pallas as pl


def kernel(start_nodes, start_times, dense_neighbor_ids, dense_neighbor_times, dense_neighbor_counts, memory_state, step_noise, restart_noise, W_restart, b_restart, time_freq, time_phase):
    raise NotImplementedError("write your pallas kernel here")


# speedup vs baseline: 1.5446x; 1.5446x over previous
import functools

import jax
import jax.numpy as jnp
from jax import lax
from jax.experimental import pallas as pl
from jax.experimental.pallas import tpu as pltpu
from jax.experimental.pallas import tpu_sc as plsc

_TEMP = 0.1
_NEG = -1e9
_SENTINEL = 2.0

try:
    _INFO = plsc.get_sparse_core_info()
    _NC, _NS = int(_INFO.num_cores), int(_INFO.num_subcores)
except Exception:
    _NC, _NS = 2, 16
_NW = _NC * _NS


def _pre_body(ids_ref, nt_ref, cnt_ref, mem_ref, wm_ref, out_ref):
    nt = nt_ref[...]
    cnt = cnt_ref[...]
    rows = nt.shape[0]
    pos = lax.broadcasted_iota(jnp.int32, (rows, 32), 1)
    tv = jnp.where(pos < cnt, nt, jnp.float32(_SENTINEL))
    memb = mem_ref[...].astype(jnp.bfloat16).astype(jnp.float32)
    wmb = wm_ref[...].astype(jnp.bfloat16).astype(jnp.float32)
    mp = jnp.sum(memb * wmb[None, :], axis=1)
    mp16 = jnp.broadcast_to(mp[:, None], (rows, 16))
    payload = jnp.concatenate([tv, mp16, jnp.zeros((rows, 48), jnp.float32)],
                              axis=1)
    out_ref[...] = jnp.concatenate(
        [ids_ref[...], lax.bitcast_convert_type(payload, jnp.int32)], axis=1)


def _build_packed(ids, ntimes, cnt, mem, wm):
    n = ntimes.shape[0]
    r = 2000
    grid = (n // r,)
    return pl.pallas_call(
        _pre_body,
        grid=grid,
        in_specs=[
            pl.BlockSpec((r, 32), lambda i: (i, 0)),
            pl.BlockSpec((r, 32), lambda i: (i, 0)),
            pl.BlockSpec((r, 1), lambda i: (i, 0)),
            pl.BlockSpec((r, 128), lambda i: (i, 0)),
            pl.BlockSpec((128,), lambda i: (0,)),
        ],
        out_specs=pl.BlockSpec((r, 128), lambda i: (i, 0)),
        out_shape=jax.ShapeDtypeStruct((n, 128), jnp.int32),
    )(ids, ntimes, cnt[:, None], mem, wm)


def _make_sc_gather(w, bpw):
    mesh = plsc.VectorSubcoreMesh(core_axis_name="c", subcore_axis_name="s")
    n_chunks = bpw // 128

    @functools.partial(
        pl.kernel,
        mesh=mesh,
        out_type=jax.ShapeDtypeStruct((w, 128), jnp.int32),
        scratch_types=[
            pltpu.VMEM((bpw,), jnp.int32),
            pltpu.VMEM((bpw, 128), jnp.int32),
            pltpu.SemaphoreType.DMA,
        ],
    )
    def gather_k(pk_hbm, cur_hbm, out_hbm, idx_v, rows_v, sem):
        wid = lax.axis_index("s") * _NC + lax.axis_index("c")
        base = wid * bpw
        pltpu.sync_copy(cur_hbm.at[pl.ds(base, bpw)], idx_v)
        copies = []
        for j in range(n_chunks):
            sl = pl.ds(j * 128, 128)
            copies.append(pltpu.async_copy(pk_hbm.at[idx_v.at[sl]],
                                           rows_v.at[sl], sem))
        for cp in copies:
            cp.wait()
        pltpu.sync_copy(rows_v, out_hbm.at[pl.ds(base, bpw)])

    return gather_k


def _step_body(pk_ref, ct_ref, sn_ref, st_ref, lp_ref,
               u_ref, rn_ref, tf_ref, tp_ref, wt_ref, b_ref,
               cur_out, ct_out, lp_out):
    packed = pk_ref[...]
    ids = packed[:, :32]
    payload = lax.bitcast_convert_type(packed[:, 32:80], jnp.float32)
    t = payload[:, :32]
    mp = jnp.max(payload[:, 32:48], axis=1)
    ct = ct_ref[...]
    rows = t.shape[0]

    valid = t < ct[:, None]
    logits = jnp.where(valid, (t - ct[:, None]) / jnp.float32(_TEMP),
                       jnp.float32(_NEG))
    score = logits + u_ref[...]

    pos = lax.broadcasted_iota(jnp.int32, (rows, 32), 1)
    smax = jnp.max(score, axis=1)
    eq = score == smax[:, None]
    choice = jnp.min(jnp.where(eq, pos, 64), axis=1)
    onehot = pos == choice[:, None]

    ml = jnp.max(logits, axis=1)
    shifted = logits - ml[:, None]
    sumexp = jnp.sum(jnp.exp(shifted), axis=1)
    chosen_shift = jnp.sum(jnp.where(onehot, shifted, 0.0), axis=1)
    step_lp = chosen_shift - jnp.log(sumexp)

    has_valid = jnp.min(t, axis=1) < ct

    nxt = jnp.sum(jnp.where(onehot, ids, 0), axis=1)
    nxt_t = jnp.sum(jnp.where(onehot, t, 0.0), axis=1)

    tenc = jnp.cos(ct[:, None] * tf_ref[...][None, :] + tp_ref[...][None, :])
    tencb = tenc.astype(jnp.bfloat16).astype(jnp.float32)
    wtb = wt_ref[...].astype(jnp.bfloat16).astype(jnp.float32)
    z = mp + jnp.sum(tencb * wtb[None, :], axis=1) + b_ref[...]
    p = jax.nn.sigmoid(z)
    do_r = rn_ref[...] < p

    lp = lp_ref[...] + jnp.where(has_valid, step_lp, 0.0)
    lp = lp + jnp.where(do_r, jnp.log(p + 1e-8), jnp.log(1.0 - p + 1e-8))

    sn = sn_ref[...]
    st = st_ref[...]
    cur_out[...] = jnp.where(do_r, sn, jnp.where(has_valid, nxt, sn))
    ct_out[...] = jnp.where(do_r, st, jnp.where(has_valid, nxt_t, st))
    lp_out[...] = lp


def _run_step(pk_rows, ct, sn, st, lp, u, rn, tf, tp, wt, b):
    w = ct.shape[0]
    bw = 2048
    grid = (w // bw,)
    blk1 = pl.BlockSpec((bw,), lambda i: (i,))
    full = lambda size: pl.BlockSpec((size,), lambda i: (0,))
    return pl.pallas_call(
        _step_body,
        grid=grid,
        in_specs=[
            pl.BlockSpec((bw, 128), lambda i: (i, 0)),
            blk1, blk1, blk1, blk1,
            pl.BlockSpec((bw, 32), lambda i: (i, 0)),
            blk1,
            full(64), full(64), full(64), full(1),
        ],
        out_specs=[blk1, blk1, blk1],
        out_shape=[
            jax.ShapeDtypeStruct((w,), jnp.int32),
            jax.ShapeDtypeStruct((w,), jnp.float32),
            jax.ShapeDtypeStruct((w,), jnp.float32),
        ],
    )(pk_rows, ct, sn, st, lp, u, rn, tf, tp, wt, b)


def kernel(start_nodes, start_times, dense_neighbor_ids, dense_neighbor_times,
           dense_neighbor_counts, memory_state, step_noise, restart_noise,
           W_restart, b_restart, time_freq, time_phase):
    b = start_nodes.shape[0]
    w = step_noise.shape[0]
    num_walks = w // b
    walk_len = step_noise.shape[1]

    wm = W_restart[:128, 0]
    wt = W_restart[128:, 0]

    pk = _build_packed(dense_neighbor_ids, dense_neighbor_times,
                       dense_neighbor_counts, memory_state, wm)

    sn = jnp.repeat(start_nodes, num_walks)
    st = jnp.repeat(start_times, num_walks)

    gum = -jnp.log(-jnp.log(jnp.clip(step_noise, 1e-7, 1.0 - 1e-7)))

    bpw = w // _NW
    gather = _make_sc_gather(w, bpw)

    cur = sn
    ct = st
    lp = jnp.zeros((w,), dtype=jnp.float32)
    walks = [cur]
    for s in range(walk_len):
        pk_rows = gather(pk, cur)
        cur, ct, lp = _run_step(pk_rows, ct, sn, st, lp,
                                gum[:, s, :], restart_noise[:, s],
                                time_freq, time_phase, wt, b_restart)
        walks.append(cur)

    return jnp.stack(walks, axis=1), lp


# speedup vs baseline: 3.6442x; 2.3593x over previous
import functools

import jax
import jax.numpy as jnp
from jax import lax
from jax.experimental import pallas as pl
from jax.experimental.pallas import tpu as pltpu
from jax.experimental.pallas import tpu_sc as plsc

_TEMP = 0.1
_NEG = -1e9
_SENTINEL = 2.0

try:
    _INFO = plsc.get_sparse_core_info()
    _NC, _NS = int(_INFO.num_cores), int(_INFO.num_subcores)
except Exception:
    _NC, _NS = 2, 16
_NW = _NC * _NS


def _pre_body(ids_ref, nt_ref, cnt_ref, mem_ref, wm_ref, out_ref):
    nt = nt_ref[...]
    cnt = cnt_ref[...]
    rows = nt.shape[0]
    pos = lax.broadcasted_iota(jnp.int32, (rows, 32), 1)
    tv = jnp.where(pos < cnt, nt, jnp.float32(_SENTINEL))
    memb = mem_ref[...].astype(jnp.bfloat16).astype(jnp.float32)
    wmb = wm_ref[...].astype(jnp.bfloat16).astype(jnp.float32)
    mp = jnp.sum(memb * wmb[None, :], axis=1)
    mp16 = jnp.broadcast_to(mp[:, None], (rows, 16))
    payload = jnp.concatenate([tv, mp16, jnp.zeros((rows, 48), jnp.float32)],
                              axis=1)
    out_ref[...] = jnp.concatenate(
        [ids_ref[...], lax.bitcast_convert_type(payload, jnp.int32)], axis=1)


def _build_packed(ids, ntimes, cnt, mem, wm):
    n = ntimes.shape[0]
    r = 2000
    grid = (n // r,)
    return pl.pallas_call(
        _pre_body,
        grid=grid,
        in_specs=[
            pl.BlockSpec((r, 32), lambda i: (i, 0)),
            pl.BlockSpec((r, 32), lambda i: (i, 0)),
            pl.BlockSpec((r, 1), lambda i: (i, 0)),
            pl.BlockSpec((r, 128), lambda i: (i, 0)),
            pl.BlockSpec((128,), lambda i: (0,)),
        ],
        out_specs=pl.BlockSpec((r, 128), lambda i: (i, 0)),
        out_shape=jax.ShapeDtypeStruct((n, 128), jnp.int32),
    )(ids, ntimes, cnt[:, None], mem, wm)


def _make_sc_gather(w, bpw):
    mesh = plsc.VectorSubcoreMesh(core_axis_name="c", subcore_axis_name="s")
    n_chunks = bpw // 128

    @functools.partial(
        pl.kernel,
        mesh=mesh,
        out_type=jax.ShapeDtypeStruct((w, 128), jnp.int32),
        scratch_types=[
            pltpu.VMEM((bpw,), jnp.int32),
            pltpu.VMEM((bpw, 128), jnp.int32),
            pltpu.SemaphoreType.DMA,
        ],
    )
    def gather_k(pk_hbm, cur_hbm, out_hbm, idx_v, rows_v, sem):
        wid = lax.axis_index("s") * _NC + lax.axis_index("c")
        base = wid * bpw
        pltpu.sync_copy(cur_hbm.at[pl.ds(base, bpw)], idx_v)
        copies = []
        for j in range(n_chunks):
            sl = pl.ds(j * 128, 128)
            copies.append(pltpu.async_copy(pk_hbm.at[idx_v.at[sl]],
                                           rows_v.at[sl], sem))
        for cp in copies:
            cp.wait()
        pltpu.sync_copy(rows_v, out_hbm.at[pl.ds(base, bpw)])

    return gather_k


def _step_body(pk_ref, ct_ref, sn_ref, st_ref, lp_ref,
               u_ref, rn_ref, tf_ref, tp_ref, wt_ref, b_ref,
               cur_out, ct_out, lp_out):
    packed = pk_ref[...]
    ids = packed[:, :32].T
    t = lax.bitcast_convert_type(packed[:, 32:64], jnp.float32).T
    mp16 = lax.bitcast_convert_type(packed[:, 64:80], jnp.float32).T
    mp = jnp.max(mp16, axis=0)
    ct = ct_ref[...]
    cols = t.shape[1]

    valid = t < ct[None, :]
    logits = jnp.where(valid, (t - ct[None, :]) / jnp.float32(_TEMP),
                       jnp.float32(_NEG))
    score = logits + u_ref[...]

    pos = lax.broadcasted_iota(jnp.int32, (32, cols), 0)
    smax = jnp.max(score, axis=0)
    eq = score == smax[None, :]
    choice = jnp.min(jnp.where(eq, pos, 64), axis=0)
    onehot = pos == choice[None, :]

    ml = jnp.max(logits, axis=0)
    shifted = logits - ml[None, :]
    sumexp = jnp.sum(jnp.exp(shifted), axis=0)
    chosen_shift = jnp.sum(jnp.where(onehot, shifted, 0.0), axis=0)
    step_lp = chosen_shift - jnp.log(sumexp)

    has_valid = jnp.min(t, axis=0) < ct

    nxt = jnp.sum(jnp.where(onehot, ids, 0), axis=0)
    nxt_t = jnp.sum(jnp.where(onehot, t, 0.0), axis=0)

    tenc = jnp.cos(ct[None, :] * tf_ref[...] + tp_ref[...])
    tencb = tenc.astype(jnp.bfloat16).astype(jnp.float32)
    wtb = wt_ref[...].astype(jnp.bfloat16).astype(jnp.float32)
    z = mp + jnp.sum(tencb * wtb, axis=0) + b_ref[...]
    p = jax.nn.sigmoid(z)
    do_r = rn_ref[...] < p

    lp = lp_ref[...] + jnp.where(has_valid, step_lp, 0.0)
    lp = lp + jnp.log(jnp.where(do_r, p, 1.0 - p) + 1e-8)

    sn = sn_ref[...]
    st = st_ref[...]
    cur_out[...] = jnp.where(do_r, sn, jnp.where(has_valid, nxt, sn))
    ct_out[...] = jnp.where(do_r, st, jnp.where(has_valid, nxt_t, st))
    lp_out[...] = lp


def _run_step(pk_rows, ct, sn, st, lp, u, rn, tf, tp, wt, b):
    w = ct.shape[0]
    bw = 2048
    grid = (w // bw,)
    blk1 = pl.BlockSpec((bw,), lambda i: (i,))
    full = lambda size: pl.BlockSpec((size,), lambda i: (0,))
    return pl.pallas_call(
        _step_body,
        grid=grid,
        in_specs=[
            pl.BlockSpec((bw, 128), lambda i: (i, 0)),
            blk1, blk1, blk1, blk1,
            pl.BlockSpec((32, bw), lambda i: (0, i)),
            blk1,
            pl.BlockSpec((64, 1), lambda i: (0, 0)),
            pl.BlockSpec((64, 1), lambda i: (0, 0)),
            pl.BlockSpec((64, 1), lambda i: (0, 0)),
            full(1),
        ],
        out_specs=[blk1, blk1, blk1],
        out_shape=[
            jax.ShapeDtypeStruct((w,), jnp.int32),
            jax.ShapeDtypeStruct((w,), jnp.float32),
            jax.ShapeDtypeStruct((w,), jnp.float32),
        ],
    )(pk_rows, ct, sn, st, lp, u, rn, tf, tp, wt, b)


def kernel(start_nodes, start_times, dense_neighbor_ids, dense_neighbor_times,
           dense_neighbor_counts, memory_state, step_noise, restart_noise,
           W_restart, b_restart, time_freq, time_phase):
    b = start_nodes.shape[0]
    w = step_noise.shape[0]
    num_walks = w // b
    walk_len = step_noise.shape[1]

    wm = W_restart[:128, 0]
    wt = W_restart[128:, 0]

    pk = _build_packed(dense_neighbor_ids, dense_neighbor_times,
                       dense_neighbor_counts, memory_state, wm)

    sn = jnp.repeat(start_nodes, num_walks)
    st = jnp.repeat(start_times, num_walks)

    gum = -jnp.log(-jnp.log(jnp.clip(step_noise, 1e-7, 1.0 - 1e-7)))
    gum_t = jnp.transpose(gum, (1, 2, 0))

    bpw = w // _NW
    gather = _make_sc_gather(w, bpw)

    cur = sn
    ct = st
    lp = jnp.zeros((w,), dtype=jnp.float32)
    walks = [cur]
    for s in range(walk_len):
        pk_rows = gather(pk, cur)
        cur, ct, lp = _run_step(pk_rows, ct, sn, st, lp,
                                gum_t[s], restart_noise[:, s],
                                time_freq[:, None], time_phase[:, None],
                                wt[:, None], b_restart)
        walks.append(cur)

    return jnp.stack(walks, axis=1), lp


# speedup vs baseline: 3.9541x; 1.0850x over previous
import functools

import jax
import jax.numpy as jnp
from jax import lax
from jax.experimental import pallas as pl
from jax.experimental.pallas import tpu as pltpu
from jax.experimental.pallas import tpu_sc as plsc

_TEMP = 0.1
_NEG = -1e9
_SENTINEL = 2.0

try:
    _INFO = plsc.get_sparse_core_info()
    _NC, _NS = int(_INFO.num_cores), int(_INFO.num_subcores)
except Exception:
    _NC, _NS = 2, 16
_NW = _NC * _NS


def _pre_body(ids_ref, nt_ref, cnt_ref, mem_ref, wm_ref, out_ref):
    nt = nt_ref[...]
    cnt = cnt_ref[...]
    rows = nt.shape[0]
    pos = lax.broadcasted_iota(jnp.int32, (rows, 32), 1)
    tv = jnp.where(pos < cnt, nt, jnp.float32(_SENTINEL))
    memb = mem_ref[...].astype(jnp.bfloat16).astype(jnp.float32)
    wmb = wm_ref[...].astype(jnp.bfloat16).astype(jnp.float32)
    mp = jnp.sum(memb * wmb[None, :], axis=1)
    mp16 = jnp.broadcast_to(mp[:, None], (rows, 16))
    payload = jnp.concatenate([tv, mp16, jnp.zeros((rows, 48), jnp.float32)],
                              axis=1)
    out_ref[...] = jnp.concatenate(
        [ids_ref[...], lax.bitcast_convert_type(payload, jnp.int32)], axis=1)


def _build_packed(ids, ntimes, cnt, mem, wm):
    n = ntimes.shape[0]
    r = 2000
    grid = (n // r,)
    return pl.pallas_call(
        _pre_body,
        grid=grid,
        in_specs=[
            pl.BlockSpec((r, 32), lambda i: (i, 0)),
            pl.BlockSpec((r, 32), lambda i: (i, 0)),
            pl.BlockSpec((r, 1), lambda i: (i, 0)),
            pl.BlockSpec((r, 128), lambda i: (i, 0)),
            pl.BlockSpec((128,), lambda i: (0,)),
        ],
        out_specs=pl.BlockSpec((r, 128), lambda i: (i, 0)),
        out_shape=jax.ShapeDtypeStruct((n, 128), jnp.int32),
    )(ids, ntimes, cnt[:, None], mem, wm)


def _make_sc_gather(w, bpw):
    mesh = plsc.VectorSubcoreMesh(core_axis_name="c", subcore_axis_name="s")
    n_full, rem = divmod(bpw, 128)
    chunks = [(i * 128, 128) for i in range(n_full)]
    if rem:
        chunks.append((n_full * 128, rem))

    @functools.partial(
        pl.kernel,
        mesh=mesh,
        out_type=jax.ShapeDtypeStruct((w, 128), jnp.int32),
        scratch_types=[
            pltpu.VMEM((bpw,), jnp.int32),
            pltpu.VMEM((bpw, 128), jnp.int32),
            pltpu.SemaphoreType.DMA,
        ],
    )
    def gather_k(pk_hbm, cur_hbm, out_hbm, idx_v, rows_v, sem):
        wid = lax.axis_index("s") * _NC + lax.axis_index("c")
        base = wid * bpw
        pltpu.sync_copy(cur_hbm.at[pl.ds(base, bpw)], idx_v)
        copies = []
        for off, size in chunks:
            sl = pl.ds(off, size)
            copies.append(pltpu.async_copy(pk_hbm.at[idx_v.at[sl]],
                                           rows_v.at[sl], sem))
        for cp in copies:
            cp.wait()
        pltpu.sync_copy(rows_v, out_hbm.at[pl.ds(base, bpw)])

    return gather_k


def _step_body(pk_ref, ct_ref, sn_ref, st_ref, lp_ref,
               u_ref, rn_ref, tf_ref, tp_ref, wt_ref, b_ref,
               cur_out, ct_out, lp_out):
    packed = pk_ref[...]
    ids = packed[:, :32].T
    t = lax.bitcast_convert_type(packed[:, 32:64], jnp.float32).T
    mp16 = lax.bitcast_convert_type(packed[:, 64:80], jnp.float32).T
    mp = jnp.max(mp16, axis=0)
    ct = ct_ref[...]
    cols = t.shape[1]

    valid = t < ct[None, :]
    logits = jnp.where(valid, (t - ct[None, :]) / jnp.float32(_TEMP),
                       jnp.float32(_NEG))
    score = logits + u_ref[...]

    pos = lax.broadcasted_iota(jnp.int32, (32, cols), 0)
    smax = jnp.max(score, axis=0)
    eq = score == smax[None, :]
    choice = jnp.min(jnp.where(eq, pos, 64), axis=0)
    onehot = pos == choice[None, :]

    ml = jnp.max(logits, axis=0)
    shifted = logits - ml[None, :]
    sumexp = jnp.sum(jnp.exp(shifted), axis=0)
    chosen_shift = jnp.sum(jnp.where(onehot, shifted, 0.0), axis=0)
    step_lp = chosen_shift - jnp.log(sumexp)

    has_valid = jnp.min(t, axis=0) < ct

    nxt = jnp.sum(jnp.where(onehot, ids, 0), axis=0)
    nxt_t = jnp.sum(jnp.where(onehot, t, 0.0), axis=0)

    tenc = jnp.cos(ct[None, :] * tf_ref[...] + tp_ref[...])
    tencb = tenc.astype(jnp.bfloat16).astype(jnp.float32)
    wtb = wt_ref[...].astype(jnp.bfloat16).astype(jnp.float32)
    z = mp + jnp.sum(tencb * wtb, axis=0) + b_ref[...]
    p = jax.nn.sigmoid(z)
    do_r = rn_ref[...] < p

    lp = lp_ref[...] + jnp.where(has_valid, step_lp, 0.0)
    lp = lp + jnp.log(jnp.where(do_r, p, 1.0 - p) + 1e-8)

    sn = sn_ref[...]
    st = st_ref[...]
    cur_out[...] = jnp.where(do_r, sn, jnp.where(has_valid, nxt, sn))
    ct_out[...] = jnp.where(do_r, st, jnp.where(has_valid, nxt_t, st))
    lp_out[...] = lp


def _run_step(pk_rows, ct, sn, st, lp, u, rn, tf, tp, wt, b):
    w = ct.shape[0]
    bw = 2048
    grid = (w // bw,)
    blk1 = pl.BlockSpec((bw,), lambda i: (i,))
    full = lambda size: pl.BlockSpec((size,), lambda i: (0,))
    return pl.pallas_call(
        _step_body,
        grid=grid,
        in_specs=[
            pl.BlockSpec((bw, 128), lambda i: (i, 0)),
            blk1, blk1, blk1, blk1,
            pl.BlockSpec((32, bw), lambda i: (0, i)),
            blk1,
            pl.BlockSpec((64, 1), lambda i: (0, 0)),
            pl.BlockSpec((64, 1), lambda i: (0, 0)),
            pl.BlockSpec((64, 1), lambda i: (0, 0)),
            full(1),
        ],
        out_specs=[blk1, blk1, blk1],
        out_shape=[
            jax.ShapeDtypeStruct((w,), jnp.int32),
            jax.ShapeDtypeStruct((w,), jnp.float32),
            jax.ShapeDtypeStruct((w,), jnp.float32),
        ],
    )(pk_rows, ct, sn, st, lp, u, rn, tf, tp, wt, b)


def kernel(start_nodes, start_times, dense_neighbor_ids, dense_neighbor_times,
           dense_neighbor_counts, memory_state, step_noise, restart_noise,
           W_restart, b_restart, time_freq, time_phase):
    b = start_nodes.shape[0]
    w = step_noise.shape[0]
    num_walks = w // b
    walk_len = step_noise.shape[1]

    wm = W_restart[:128, 0]
    wt = W_restart[128:, 0]

    pk = _build_packed(dense_neighbor_ids, dense_neighbor_times,
                       dense_neighbor_counts, memory_state, wm)

    sn = jnp.repeat(start_nodes, num_walks)
    st = jnp.repeat(start_times, num_walks)

    gum = -jnp.log(-jnp.log(jnp.clip(step_noise, 1e-7, 1.0 - 1e-7)))
    gum_t = jnp.transpose(gum, (1, 2, 0))

    nh = 2
    hw = w // nh
    gather = _make_sc_gather(hw, hw // _NW)

    cur = [sn[h * hw:(h + 1) * hw] for h in range(nh)]
    snh = list(cur)
    sth = [st[h * hw:(h + 1) * hw] for h in range(nh)]
    ct = list(sth)
    lp = [jnp.zeros((hw,), dtype=jnp.float32) for _ in range(nh)]
    walks = [[c] for c in cur]
    for s in range(walk_len):
        rows = [gather(pk, cur[h]) for h in range(nh)]
        for h in range(nh):
            cur[h], ct[h], lp[h] = _run_step(
                rows[h], ct[h], snh[h], sth[h], lp[h],
                gum_t[s, :, h * hw:(h + 1) * hw],
                restart_noise[h * hw:(h + 1) * hw, s],
                time_freq[:, None], time_phase[:, None],
                wt[:, None], b_restart)
            walks[h].append(cur[h])

    return (jnp.concatenate([jnp.stack(wk, axis=1) for wk in walks], axis=0),
            jnp.concatenate(lp))


# speedup vs baseline: 4.0639x; 1.0278x over previous
import functools

import jax
import jax.numpy as jnp
from jax import lax
from jax.experimental import pallas as pl
from jax.experimental.pallas import tpu as pltpu
from jax.experimental.pallas import tpu_sc as plsc

_TEMP = 0.1
_NEG = -1e9
_SENTINEL = 2.0

try:
    _INFO = plsc.get_sparse_core_info()
    _NC, _NS = int(_INFO.num_cores), int(_INFO.num_subcores)
except Exception:
    _NC, _NS = 2, 16
_NW = _NC * _NS


def _pre_body(ids_ref, nt_ref, cnt_ref, mem_ref, wm_ref, out_ref):
    nt = nt_ref[...]
    cnt = cnt_ref[...]
    rows = nt.shape[0]
    pos = lax.broadcasted_iota(jnp.int32, (rows, 32), 1)
    tv = jnp.where(pos < cnt, nt, jnp.float32(_SENTINEL))
    memb = mem_ref[...].astype(jnp.bfloat16).astype(jnp.float32)
    wmb = wm_ref[...].astype(jnp.bfloat16).astype(jnp.float32)
    mp = jnp.sum(memb * wmb[None, :], axis=1)
    mp16 = jnp.broadcast_to(mp[:, None], (rows, 16))
    payload = jnp.concatenate([tv, mp16, jnp.zeros((rows, 48), jnp.float32)],
                              axis=1)
    out_ref[...] = jnp.concatenate(
        [ids_ref[...], lax.bitcast_convert_type(payload, jnp.int32)], axis=1)


def _build_packed(ids, ntimes, cnt, mem, wm):
    n = ntimes.shape[0]
    r = 2000
    grid = (n // r,)
    return pl.pallas_call(
        _pre_body,
        grid=grid,
        in_specs=[
            pl.BlockSpec((r, 32), lambda i: (i, 0)),
            pl.BlockSpec((r, 32), lambda i: (i, 0)),
            pl.BlockSpec((r, 1), lambda i: (i, 0)),
            pl.BlockSpec((r, 128), lambda i: (i, 0)),
            pl.BlockSpec((128,), lambda i: (0,)),
        ],
        out_specs=pl.BlockSpec((r, 128), lambda i: (i, 0)),
        out_shape=jax.ShapeDtypeStruct((n, 128), jnp.int32),
    )(ids, ntimes, cnt[:, None], mem, wm)


def _make_sc_gather(w, bpw):
    mesh = plsc.VectorSubcoreMesh(core_axis_name="c", subcore_axis_name="s")
    n_full, rem = divmod(bpw, 128)
    chunks = [(i * 128, 128) for i in range(n_full)]
    if rem:
        chunks.append((n_full * 128, rem))

    @functools.partial(
        pl.kernel,
        mesh=mesh,
        out_type=jax.ShapeDtypeStruct((w, 128), jnp.int32),
        scratch_types=[
            pltpu.VMEM((bpw,), jnp.int32),
            pltpu.VMEM((bpw, 128), jnp.int32),
            pltpu.SemaphoreType.DMA,
        ],
    )
    def gather_k(pk_hbm, cur_hbm, out_hbm, idx_v, rows_v, sem):
        wid = lax.axis_index("s") * _NC + lax.axis_index("c")
        base = wid * bpw
        pltpu.sync_copy(cur_hbm.at[pl.ds(base, bpw)], idx_v)
        copies = []
        for off, size in chunks:
            sl = pl.ds(off, size)
            copies.append(pltpu.async_copy(pk_hbm.at[idx_v.at[sl]],
                                           rows_v.at[sl], sem))
        for cp in copies:
            cp.wait()
        pltpu.sync_copy(rows_v, out_hbm.at[pl.ds(base, bpw)])

    return gather_k


def _step_body(pk_ref, ct_ref, sn_ref, st_ref, lp_ref,
               u_ref, rn_ref, tf_ref, tp_ref, wt_ref, b_ref,
               cur_out, ct_out, lp_out):
    packed = pk_ref[...]
    ids = packed[:, :32].T
    t = lax.bitcast_convert_type(packed[:, 32:64], jnp.float32).T
    mp16 = lax.bitcast_convert_type(packed[:, 64:80], jnp.float32).T
    mp = jnp.max(mp16, axis=0)
    ct = ct_ref[...]
    cols = t.shape[1]

    valid = t < ct[None, :]
    logits = jnp.where(valid, (t - ct[None, :]) / jnp.float32(_TEMP),
                       jnp.float32(_NEG))
    score = logits + u_ref[...]

    pos = lax.broadcasted_iota(jnp.int32, (32, cols), 0)
    smax = jnp.max(score, axis=0)
    eq = score == smax[None, :]
    choice = jnp.min(jnp.where(eq, pos, 64), axis=0)
    onehot = pos == choice[None, :]

    ml = jnp.max(logits, axis=0)
    shifted = logits - ml[None, :]
    sumexp = jnp.sum(jnp.exp(shifted), axis=0)
    chosen_shift = jnp.sum(jnp.where(onehot, shifted, 0.0), axis=0)
    step_lp = chosen_shift - jnp.log(sumexp)

    has_valid = jnp.min(t, axis=0) < ct

    nxt = jnp.sum(jnp.where(onehot, ids, 0), axis=0)
    nxt_t = jnp.sum(jnp.where(onehot, t, 0.0), axis=0)

    tenc = jnp.cos(ct[None, :] * tf_ref[...] + tp_ref[...])
    tencb = tenc.astype(jnp.bfloat16).astype(jnp.float32)
    wtb = wt_ref[...].astype(jnp.bfloat16).astype(jnp.float32)
    z = mp + jnp.sum(tencb * wtb, axis=0) + b_ref[...]
    p = jax.nn.sigmoid(z)
    do_r = rn_ref[...] < p

    lp = lp_ref[...] + jnp.where(has_valid, step_lp, 0.0)
    lp = lp + jnp.log(jnp.where(do_r, p, 1.0 - p) + 1e-8)

    sn = sn_ref[...]
    st = st_ref[...]
    cur_out[...] = jnp.where(do_r, sn, jnp.where(has_valid, nxt, sn))
    ct_out[...] = jnp.where(do_r, st, jnp.where(has_valid, nxt_t, st))
    lp_out[...] = lp


def _run_step(pk_rows, ct, sn, st, lp, u, rn, tf, tp, wt, b):
    w = ct.shape[0]
    bw = 2048
    grid = (w // bw,)
    blk1 = pl.BlockSpec((bw,), lambda i: (i,))
    full = lambda size: pl.BlockSpec((size,), lambda i: (0,))
    return pl.pallas_call(
        _step_body,
        grid=grid,
        in_specs=[
            pl.BlockSpec((bw, 128), lambda i: (i, 0)),
            blk1, blk1, blk1, blk1,
            pl.BlockSpec((32, bw), lambda i: (0, i)),
            blk1,
            pl.BlockSpec((64, 1), lambda i: (0, 0)),
            pl.BlockSpec((64, 1), lambda i: (0, 0)),
            pl.BlockSpec((64, 1), lambda i: (0, 0)),
            full(1),
        ],
        out_specs=[blk1, blk1, blk1],
        out_shape=[
            jax.ShapeDtypeStruct((w,), jnp.int32),
            jax.ShapeDtypeStruct((w,), jnp.float32),
            jax.ShapeDtypeStruct((w,), jnp.float32),
        ],
    )(pk_rows, ct, sn, st, lp, u, rn, tf, tp, wt, b)


def kernel(start_nodes, start_times, dense_neighbor_ids, dense_neighbor_times,
           dense_neighbor_counts, memory_state, step_noise, restart_noise,
           W_restart, b_restart, time_freq, time_phase):
    b = start_nodes.shape[0]
    w = step_noise.shape[0]
    num_walks = w // b
    walk_len = step_noise.shape[1]

    wm = W_restart[:128, 0]
    wt = W_restart[128:, 0]

    pk = _build_packed(dense_neighbor_ids, dense_neighbor_times,
                       dense_neighbor_counts, memory_state, wm)

    sn = jnp.repeat(start_nodes, num_walks)
    st = jnp.repeat(start_times, num_walks)

    def _gum(h, s, hw):
        blk = step_noise[h * hw:(h + 1) * hw, s, :]
        return (-jnp.log(-jnp.log(jnp.clip(blk, 1e-7, 1.0 - 1e-7)))).T

    nh = 2
    hw = w // nh
    gather = _make_sc_gather(hw, hw // _NW)

    cur = [sn[h * hw:(h + 1) * hw] for h in range(nh)]
    snh = list(cur)
    sth = [st[h * hw:(h + 1) * hw] for h in range(nh)]
    ct = list(sth)
    lp = [jnp.zeros((hw,), dtype=jnp.float32) for _ in range(nh)]
    walks = [[c] for c in cur]
    for s in range(walk_len):
        rows = [gather(pk, cur[h]) for h in range(nh)]
        for h in range(nh):
            cur[h], ct[h], lp[h] = _run_step(
                rows[h], ct[h], snh[h], sth[h], lp[h],
                _gum(h, s, hw),
                restart_noise[h * hw:(h + 1) * hw, s],
                time_freq[:, None], time_phase[:, None],
                wt[:, None], b_restart)
            walks[h].append(cur[h])

    return (jnp.concatenate([jnp.stack(wk, axis=1) for wk in walks], axis=0),
            jnp.concatenate(lp))
